# baseline (speedup 1.0000x reference)
import functools

import jax
import jax.numpy as jnp
from jax import lax
from jax.experimental import pallas as pl
from jax.experimental.pallas import tpu as pltpu
from jax.experimental.pallas import tpu_sc as plsc

ROWS = 8192
DIM = 4096
NC = 2
NS = 16
NW = NC * NS
ROWS_PER_W = ROWS // NW
R = 8
NBLK = ROWS_PER_W // R
NCHUNK = DIM // 16


def _body(x_hbm, perm_hbm, out_hbm, perm_v, x_v, out_v):
    wid = lax.axis_index("s") * NC + lax.axis_index("c")
    base0 = wid * (ROWS_PER_W * DIM)
    pltpu.sync_copy(perm_hbm, perm_v)

    def block(b, _):
        base = base0 + b * (R * DIM)
        pltpu.sync_copy(x_hbm.at[pl.ds(base, R * DIM)], x_v)

        def chunk(jg, _):
            col = jg * 16
            pv = perm_v[pl.ds(col, 16)]
            for i in range(R):
                out_v[pl.ds(i * DIM + col, 16)] = plsc.load_gather(
                    x_v, [pv + (i * DIM)])
            return 0

        lax.fori_loop(0, NCHUNK, chunk, 0)
        pltpu.sync_copy(out_v, out_hbm.at[pl.ds(base, R * DIM)])
        return 0

    lax.fori_loop(0, NBLK, block, 0)


@jax.jit
def kernel(x, perm):
    perm32 = perm.astype(jnp.int32)
    x_flat = x.reshape(ROWS * DIM)
    mesh = plsc.VectorSubcoreMesh(core_axis_name="c", subcore_axis_name="s")
    f = pl.kernel(
        _body,
        out_type=jax.ShapeDtypeStruct((ROWS * DIM,), jnp.float32),
        mesh=mesh,
        scratch_types=[
            pltpu.VMEM((DIM,), jnp.int32),
            pltpu.VMEM((R * DIM,), jnp.float32),
            pltpu.VMEM((R * DIM,), jnp.float32),
        ],
        compiler_params=pltpu.CompilerParams(needs_layout_passes=False),
    )
    return f(x_flat, perm32).reshape(ROWS, DIM)


# speedup vs baseline: 1.1164x; 1.1164x over previous
import functools

import jax
import jax.numpy as jnp
from jax import lax
from jax.experimental import pallas as pl
from jax.experimental.pallas import tpu as pltpu
from jax.experimental.pallas import tpu_sc as plsc

ROWS = 8192
DIM = 4096
NC = 2
NS = 16
NW = NC * NS
ROWS_PER_W = ROWS // NW
R = 4
BLK = R * DIM
NBLK = ROWS_PER_W // R
NCHUNK = DIM // 16


def _body(x_hbm, perm_hbm, out_hbm, perm_v, x0, x1, o0, o1,
          sx0, sx1, so0, so1):
    xbufs = (x0, x1)
    obufs = (o0, o1)
    sxs = (sx0, sx1)
    sos = (so0, so1)
    wid = lax.axis_index("s") * NC + lax.axis_index("c")
    base0 = wid * (ROWS_PER_W * DIM)
    pltpu.sync_copy(perm_hbm, perm_v)

    for p in range(2):
        pltpu.async_copy(x_hbm.at[pl.ds(base0 + p * BLK, BLK)],
                         xbufs[p], sxs[p])

    @pl.loop(0, NBLK, step=2)
    def _blocks(bb):
        for p in range(2):
            b = bb + p
            base = base0 + b * BLK
            xv, ov = xbufs[p], obufs[p]
            pltpu.make_async_copy(
                x_hbm.at[pl.ds(base, BLK)], xv, sxs[p]).wait()

            @pl.when(b >= 2)
            def _wait_out():
                pltpu.make_async_copy(
                    ov, out_hbm.at[pl.ds(base, BLK)], sos[p]).wait()

            @pl.loop(0, NCHUNK, unroll=2)
            def _chunk(jg):
                col = jg * 16
                pv = perm_v[pl.ds(col, 16)]
                for i in range(R):
                    ov[pl.ds(i * DIM + col, 16)] = plsc.load_gather(
                        xv, [pv + (i * DIM)])

            pltpu.async_copy(ov, out_hbm.at[pl.ds(base, BLK)], sos[p])

            @pl.when(b + 2 < NBLK)
            def _next_x():
                pltpu.async_copy(
                    x_hbm.at[pl.ds(base + 2 * BLK, BLK)], xv, sxs[p])

    for p in range(2):
        pltpu.make_async_copy(
            obufs[p], out_hbm.at[pl.ds(base0, BLK)], sos[p]).wait()


@jax.jit
def kernel(x, perm):
    perm32 = perm.astype(jnp.int32)
    x_flat = x.reshape(ROWS * DIM)
    mesh = plsc.VectorSubcoreMesh(core_axis_name="c", subcore_axis_name="s")
    f = pl.kernel(
        _body,
        out_type=jax.ShapeDtypeStruct((ROWS * DIM,), jnp.float32),
        mesh=mesh,
        scratch_types=[
            pltpu.VMEM((DIM,), jnp.int32),
            pltpu.VMEM((BLK,), jnp.float32),
            pltpu.VMEM((BLK,), jnp.float32),
            pltpu.VMEM((BLK,), jnp.float32),
            pltpu.VMEM((BLK,), jnp.float32),
            pltpu.SemaphoreType.DMA,
            pltpu.SemaphoreType.DMA,
            pltpu.SemaphoreType.DMA,
            pltpu.SemaphoreType.DMA,
        ],
        compiler_params=pltpu.CompilerParams(needs_layout_passes=False),
    )
    return f(x_flat, perm32).reshape(ROWS, DIM)


# speedup vs baseline: 1.9688x; 1.7635x over previous
import functools

import jax
import jax.numpy as jnp
from jax import lax
from jax.experimental import pallas as pl
from jax.experimental.pallas import tpu as pltpu
from jax.experimental.pallas import tpu_sc as plsc

ROWS = 8192
DIM = 4096
NC = 2
NS = 16
NW = NC * NS
ROWS_PER_W = ROWS // NW
R = 4
BLK = R * DIM
NBLK = ROWS_PER_W // R
NCHUNK = DIM // 16


def _body(x_hbm, perm_hbm, out_hbm, perm_v, x0, x1, o0, o1,
          sx0, sx1, so0, so1):
    xbufs = (x0, x1)
    obufs = (o0, o1)
    sxs = (sx0, sx1)
    sos = (so0, so1)
    wid = lax.axis_index("s") * NC + lax.axis_index("c")
    base0 = wid * (ROWS_PER_W * DIM)
    pltpu.sync_copy(perm_hbm, perm_v)

    for p in range(2):
        pltpu.async_copy(x_hbm.at[pl.ds(base0 + p * BLK, BLK)],
                         xbufs[p], sxs[p])

    @pl.loop(0, NBLK, step=2)
    def _blocks(bb):
        for p in range(2):
            b = bb + p
            base = base0 + b * BLK
            xv, ov = xbufs[p], obufs[p]
            pltpu.make_async_copy(
                x_hbm.at[pl.ds(base, BLK)], xv, sxs[p]).wait()

            @pl.when(b >= 2)
            def _wait_out():
                pltpu.make_async_copy(
                    ov, out_hbm.at[pl.ds(base, BLK)], sos[p]).wait()

            @plsc.parallel_loop(0, NCHUNK, unroll=4)
            def _chunk(jg):
                col = jg * 16
                pv = perm_v[pl.ds(col, 16)]
                for i in range(R):
                    ov[pl.ds(i * DIM + col, 16)] = plsc.load_gather(
                        xv, [pv + (i * DIM)])

            pltpu.async_copy(ov, out_hbm.at[pl.ds(base, BLK)], sos[p])

            @pl.when(b + 2 < NBLK)
            def _next_x():
                pltpu.async_copy(
                    x_hbm.at[pl.ds(base + 2 * BLK, BLK)], xv, sxs[p])

    for p in range(2):
        pltpu.make_async_copy(
            obufs[p], out_hbm.at[pl.ds(base0, BLK)], sos[p]).wait()


@jax.jit
def kernel(x, perm):
    perm32 = perm.astype(jnp.int32)
    x_flat = x.reshape(ROWS * DIM)
    mesh = plsc.VectorSubcoreMesh(core_axis_name="c", subcore_axis_name="s")
    f = pl.kernel(
        _body,
        out_type=jax.ShapeDtypeStruct((ROWS * DIM,), jnp.float32),
        mesh=mesh,
        scratch_types=[
            pltpu.VMEM((DIM,), jnp.int32),
            pltpu.VMEM((BLK,), jnp.float32),
            pltpu.VMEM((BLK,), jnp.float32),
            pltpu.VMEM((BLK,), jnp.float32),
            pltpu.VMEM((BLK,), jnp.float32),
            pltpu.SemaphoreType.DMA,
            pltpu.SemaphoreType.DMA,
            pltpu.SemaphoreType.DMA,
            pltpu.SemaphoreType.DMA,
        ],
        compiler_params=pltpu.CompilerParams(needs_layout_passes=False),
    )
    return f(x_flat, perm32).reshape(ROWS, DIM)


# speedup vs baseline: 1.9697x; 1.0005x over previous
import functools

import jax
import jax.numpy as jnp
from jax import lax
from jax.experimental import pallas as pl
from jax.experimental.pallas import tpu as pltpu
from jax.experimental.pallas import tpu_sc as plsc

ROWS = 8192
DIM = 4096
NC = 2
NS = 16
NW = NC * NS
ROWS_PER_W = ROWS // NW
R = 4
BLK = R * DIM
NBLK = ROWS_PER_W // R
NCHUNK = DIM // 16


def _body(x_hbm, perm_hbm, out_hbm, perm_v, x0, x1, o0, o1,
          sx0, sx1, so0, so1):
    xbufs = (x0, x1)
    obufs = (o0, o1)
    sxs = (sx0, sx1)
    sos = (so0, so1)
    wid = lax.axis_index("s") * NC + lax.axis_index("c")
    base0 = wid * (ROWS_PER_W * DIM)
    pltpu.sync_copy(perm_hbm, perm_v)

    for p in range(2):
        pltpu.async_copy(x_hbm.at[pl.ds(base0 + p * BLK, BLK)],
                         xbufs[p], sxs[p])

    @pl.loop(0, NBLK, step=2)
    def _blocks(bb):
        for p in range(2):
            b = bb + p
            base = base0 + b * BLK
            xv, ov = xbufs[p], obufs[p]
            pltpu.make_async_copy(
                x_hbm.at[pl.ds(base, BLK)], xv, sxs[p]).wait()

            @pl.when(b >= 2)
            def _wait_out():
                pltpu.make_async_copy(
                    ov, out_hbm.at[pl.ds(base, BLK)], sos[p]).wait()

            @plsc.parallel_loop(0, NCHUNK, unroll=8)
            def _chunk(jg):
                col = jg * 16
                pv = perm_v[pl.ds(col, 16)]
                for i in range(R):
                    ov[pl.ds(i * DIM + col, 16)] = plsc.load_gather(
                        xv, [pv + (i * DIM)])

            pltpu.async_copy(ov, out_hbm.at[pl.ds(base, BLK)], sos[p])

            @pl.when(b + 2 < NBLK)
            def _next_x():
                pltpu.async_copy(
                    x_hbm.at[pl.ds(base + 2 * BLK, BLK)], xv, sxs[p])

    for p in range(2):
        pltpu.make_async_copy(
            obufs[p], out_hbm.at[pl.ds(base0, BLK)], sos[p]).wait()


@jax.jit
def kernel(x, perm):
    perm32 = perm.astype(jnp.int32)
    x_flat = x.reshape(ROWS * DIM)
    mesh = plsc.VectorSubcoreMesh(core_axis_name="c", subcore_axis_name="s")
    f = pl.kernel(
        _body,
        out_type=jax.ShapeDtypeStruct((ROWS * DIM,), jnp.float32),
        mesh=mesh,
        scratch_types=[
            pltpu.VMEM((DIM,), jnp.int32),
            pltpu.VMEM((BLK,), jnp.float32),
            pltpu.VMEM((BLK,), jnp.float32),
            pltpu.VMEM((BLK,), jnp.float32),
            pltpu.VMEM((BLK,), jnp.float32),
            pltpu.SemaphoreType.DMA,
            pltpu.SemaphoreType.DMA,
            pltpu.SemaphoreType.DMA,
            pltpu.SemaphoreType.DMA,
        ],
        compiler_params=pltpu.CompilerParams(needs_layout_passes=False),
    )
    return f(x_flat, perm32).reshape(ROWS, DIM)


# speedup vs baseline: 5.7406x; 2.9145x over previous
import functools

import jax
import jax.numpy as jnp
from jax import lax
from jax.experimental import pallas as pl
from jax.experimental.pallas import tpu as pltpu
from jax.experimental.pallas import tpu_sc as plsc

ROWS = 8192
DIM = 4096
NC = 2
NS = 16
NW = NC * NS
ROWS_PER_W = ROWS // NW
R = 4
NBLK = ROWS_PER_W // R
NCHUNK = DIM // 16


def _body(x_hbm, perm_hbm, out_hbm, perm_v, x0, x1, o0, o1,
          sx0, sx1, so0, so1):
    xbufs = (x0, x1)
    obufs = (o0, o1)
    sxs = (sx0, sx1)
    sos = (so0, so1)
    wid = lax.axis_index("s") * NC + lax.axis_index("c")
    row0 = wid * ROWS_PER_W
    pltpu.sync_copy(perm_hbm, perm_v)

    for p in range(2):
        pltpu.async_copy(x_hbm.at[pl.ds(row0 + p * R, R)], xbufs[p], sxs[p])

    @pl.loop(0, NBLK, step=2)
    def _blocks(bb):
        for p in range(2):
            b = bb + p
            base = row0 + b * R
            xv, ov = xbufs[p], obufs[p]
            pltpu.make_async_copy(
                x_hbm.at[pl.ds(base, R)], xv, sxs[p]).wait()

            @pl.when(b >= 2)
            def _wait_out():
                pltpu.make_async_copy(
                    ov, out_hbm.at[pl.ds(base, R)], sos[p]).wait()

            @plsc.parallel_loop(0, NCHUNK, unroll=4)
            def _chunk(jg):
                col = jg * 16
                pv = perm_v[pl.ds(col, 16)]
                for i in range(R):
                    row = jnp.full((16,), i, dtype=jnp.int32)
                    ov[i, pl.ds(col, 16)] = plsc.load_gather(xv, [row, pv])

            pltpu.async_copy(ov, out_hbm.at[pl.ds(base, R)], sos[p])

            @pl.when(b + 2 < NBLK)
            def _next_x():
                pltpu.async_copy(
                    x_hbm.at[pl.ds(base + 2 * R, R)], xv, sxs[p])

    for p in range(2):
        pltpu.make_async_copy(
            obufs[p], out_hbm.at[pl.ds(row0, R)], sos[p]).wait()


@jax.jit
def kernel(x, perm):
    perm32 = perm.astype(jnp.int32)
    mesh = plsc.VectorSubcoreMesh(core_axis_name="c", subcore_axis_name="s")
    f = pl.kernel(
        _body,
        out_type=jax.ShapeDtypeStruct((ROWS, DIM), jnp.float32),
        mesh=mesh,
        scratch_types=[
            pltpu.VMEM((DIM,), jnp.int32),
            pltpu.VMEM((R, DIM), jnp.float32),
            pltpu.VMEM((R, DIM), jnp.float32),
            pltpu.VMEM((R, DIM), jnp.float32),
            pltpu.VMEM((R, DIM), jnp.float32),
            pltpu.SemaphoreType.DMA,
            pltpu.SemaphoreType.DMA,
            pltpu.SemaphoreType.DMA,
            pltpu.SemaphoreType.DMA,
        ],
        compiler_params=pltpu.CompilerParams(needs_layout_passes=False),
    )
    return f(x, perm32)


# speedup vs baseline: 5.9068x; 1.0290x over previous
import functools

import jax
import jax.numpy as jnp
from jax import lax
from jax.experimental import pallas as pl
from jax.experimental.pallas import tpu as pltpu
from jax.experimental.pallas import tpu_sc as plsc

ROWS = 8192
DIM = 4096
HALF = DIM // 2
NC = 2
NS = 16
NW = NC * NS
ROWS_PER_W = ROWS // NW
R = 8
NBLK = ROWS_PER_W // R
HCHUNK = HALF // 16


def _body(x_hbm, perm_hbm, out_hbm, perm_v, x0, x1, o0, o1,
          sx0, sx1, so0, so1):
    xbufs = (x0, x1)
    obufs = (o0, o1)
    sxs = (sx0, sx1)
    sos = (so0, so1)
    wid = lax.axis_index("s") * NC + lax.axis_index("c")
    row0 = wid * ROWS_PER_W
    pltpu.sync_copy(perm_hbm, perm_v)

    for p in range(2):
        pltpu.async_copy(x_hbm.at[pl.ds(row0 + p * R, R)], xbufs[p], sxs[p])

    rowvecs = [jnp.full((16,), i, dtype=jnp.int32) for i in range(R)]

    @pl.loop(0, NBLK)
    def _blocks(b):
        p = lax.rem(b, 2)
        base = row0 + b * R

        def run(xv):
            for h in range(2):
                ov, so = obufs[h], sos[h]

                @pl.when(b >= 1)
                def _wait_out():
                    pltpu.make_async_copy(
                        ov, out_hbm.at[pl.ds(base, R), pl.ds(h * HALF, HALF)],
                        so).wait()

                @plsc.parallel_loop(0, HCHUNK, unroll=4)
                def _chunk(jg):
                    col = jg * 16
                    pv = perm_v[pl.ds(h * HALF + col, 16)]
                    for i in range(R):
                        ov[i, pl.ds(col, 16)] = plsc.load_gather(
                            xv, [rowvecs[i], pv])

                pltpu.async_copy(
                    ov, out_hbm.at[pl.ds(base, R), pl.ds(h * HALF, HALF)], so)

            @pl.when(b + 2 < NBLK)
            def _next_x():
                pltpu.async_copy(
                    x_hbm.at[pl.ds(base + 2 * R, R)], xv,
                    sxs[0] if xv is xbufs[0] else sxs[1])

        @pl.when(p == 0)
        def _even():
            pltpu.make_async_copy(
                x_hbm.at[pl.ds(base, R)], xbufs[0], sxs[0]).wait()
            run(xbufs[0])

        @pl.when(p == 1)
        def _odd():
            pltpu.make_async_copy(
                x_hbm.at[pl.ds(base, R)], xbufs[1], sxs[1]).wait()
            run(xbufs[1])

    for h in range(2):
        pltpu.make_async_copy(
            obufs[h], out_hbm.at[pl.ds(row0, R), pl.ds(h * HALF, HALF)],
            sos[h]).wait()


@jax.jit
def kernel(x, perm):
    perm32 = perm.astype(jnp.int32)
    mesh = plsc.VectorSubcoreMesh(core_axis_name="c", subcore_axis_name="s")
    f = pl.kernel(
        _body,
        out_type=jax.ShapeDtypeStruct((ROWS, DIM), jnp.float32),
        mesh=mesh,
        scratch_types=[
            pltpu.VMEM((DIM,), jnp.int32),
            pltpu.VMEM((R, DIM), jnp.float32),
            pltpu.VMEM((R, DIM), jnp.float32),
            pltpu.VMEM((R, HALF), jnp.float32),
            pltpu.VMEM((R, HALF), jnp.float32),
            pltpu.SemaphoreType.DMA,
            pltpu.SemaphoreType.DMA,
            pltpu.SemaphoreType.DMA,
            pltpu.SemaphoreType.DMA,
        ],
        compiler_params=pltpu.CompilerParams(needs_layout_passes=False),
    )
    return f(x, perm32)


# speedup vs baseline: 5.9234x; 1.0028x over previous
import functools

import jax
import jax.numpy as jnp
from jax import lax
from jax.experimental import pallas as pl
from jax.experimental.pallas import tpu as pltpu
from jax.experimental.pallas import tpu_sc as plsc

ROWS = 8192
DIM = 4096
HALF = DIM // 2
NC = 2
NS = 16
NW = NC * NS
ROWS_PER_W = ROWS // NW
R = 4
NBLK = ROWS_PER_W // R
HCHUNK = HALF // 16


def _body(x_hbm, perm_hbm, out_hbm, perm_v,
          x0, x1, x2, x3, o0, o1, o2, o3,
          sx0, sx1, sx2, sx3, so0, so1, so2, so3):
    xbufs = (x0, x1, x2, x3)
    obufs = (o0, o1, o2, o3)
    sxs = (sx0, sx1, sx2, sx3)
    sos = (so0, so1, so2, so3)
    wid = lax.axis_index("s") * NC + lax.axis_index("c")
    row0 = wid * ROWS_PER_W
    pltpu.sync_copy(perm_hbm, perm_v)

    for p in range(3):
        pltpu.async_copy(x_hbm.at[pl.ds(row0 + p * R, R)], xbufs[p], sxs[p])

    rowvecs = [jnp.full((16,), i, dtype=jnp.int32) for i in range(R)]

    @pl.loop(0, NBLK)
    def _blocks(b):
        base = row0 + b * R
        p4 = lax.rem(b, 4)
        for k in range(4):
            @pl.when(p4 == k)
            def _parity(k=k):
                xv = xbufs[k]
                pltpu.make_async_copy(
                    x_hbm.at[pl.ds(base, R)], xv, sxs[k]).wait()

                @pl.when(b + 3 < NBLK)
                def _prefetch():
                    pltpu.async_copy(
                        x_hbm.at[pl.ds(base + 3 * R, R)],
                        xbufs[(k + 3) % 4], sxs[(k + 3) % 4])

                for h in range(2):
                    hb = (2 * k + h) % 4
                    ov, so = obufs[hb], sos[hb]
                    dst = out_hbm.at[pl.ds(base, R), pl.ds(h * HALF, HALF)]

                    @pl.when(b >= 2)
                    def _wait_out():
                        pltpu.make_async_copy(ov, dst, so).wait()

                    @plsc.parallel_loop(0, HCHUNK, unroll=4)
                    def _chunk(jg):
                        col = jg * 16
                        pv = perm_v[pl.ds(h * HALF + col, 16)]
                        for i in range(R):
                            ov[i, pl.ds(col, 16)] = plsc.load_gather(
                                xv, [rowvecs[i], pv])

                    pltpu.async_copy(ov, dst, so)

    for hb in range(4):
        pltpu.make_async_copy(
            obufs[hb], out_hbm.at[pl.ds(row0, R), pl.ds(0, HALF)],
            sos[hb]).wait()


@jax.jit
def kernel(x, perm):
    perm32 = perm.astype(jnp.int32)
    mesh = plsc.VectorSubcoreMesh(core_axis_name="c", subcore_axis_name="s")
    f = pl.kernel(
        _body,
        out_type=jax.ShapeDtypeStruct((ROWS, DIM), jnp.float32),
        mesh=mesh,
        scratch_types=[
            pltpu.VMEM((DIM,), jnp.int32),
            pltpu.VMEM((R, DIM), jnp.float32),
            pltpu.VMEM((R, DIM), jnp.float32),
            pltpu.VMEM((R, DIM), jnp.float32),
            pltpu.VMEM((R, DIM), jnp.float32),
            pltpu.VMEM((R, HALF), jnp.float32),
            pltpu.VMEM((R, HALF), jnp.float32),
            pltpu.VMEM((R, HALF), jnp.float32),
            pltpu.VMEM((R, HALF), jnp.float32),
            pltpu.SemaphoreType.DMA,
            pltpu.SemaphoreType.DMA,
            pltpu.SemaphoreType.DMA,
            pltpu.SemaphoreType.DMA,
            pltpu.SemaphoreType.DMA,
            pltpu.SemaphoreType.DMA,
            pltpu.SemaphoreType.DMA,
            pltpu.SemaphoreType.DMA,
        ],
        compiler_params=pltpu.CompilerParams(needs_layout_passes=False),
    )
    return f(x, perm32)
